# baseline (speedup 1.0000x reference)
import jax
import jax.numpy as jnp
from jax import lax
from jax.experimental import pallas as pl
from jax.experimental.pallas import tpu as pltpu
from jax.experimental.pallas import tpu_sc as plsc

N = 10000
E = 320000
IN_DIM = 128
HIDDEN = 128
LATENT = 64

NC = 2
NS = 16
BLK = 128
NBLKS = E // BLK
BASE_BLKS = NBLKS // NS
EXTRA = NBLKS - BASE_BLKS * NS
MAXB = BASE_BLKS + 1
ROWS_A = 624
ROWS_LAST = N - ROWS_A * (NS - 1)

_MESH = plsc.VectorSubcoreMesh(
    core_axis_name="c", subcore_axis_name="s", num_cores=NC, num_subcores=NS)


def _make_segsum(DH, with_deg):
  NBUF = 4
  out_type = [jax.ShapeDtypeStruct((NC, N, DH), jnp.float32)]
  if with_deg:
    out_type.append(jax.ShapeDtypeStruct((NC, N, 16), jnp.float32))
  scratch = [
      pltpu.VMEM((MAXB, BLK), jnp.int32),
      pltpu.VMEM((MAXB, BLK), jnp.int32),
  ]
  scratch += [pltpu.VMEM((BLK, DH), jnp.float32) for _ in range(NBUF)]
  scratch += [
      pltpu.VMEM_SHARED((N, DH), jnp.float32),
  ]
  scratch += [pltpu.SemaphoreType.DMA for _ in range(2 * NBUF)]
  if with_deg:
    scratch += [
        pltpu.VMEM((BLK, 16), jnp.float32),
        pltpu.VMEM_SHARED((N, 16), jnp.float32),
    ]

  def body(*refs):
    if with_deg:
      (p_hbm, ei_hbm, z_hbm, ones_hbm, z1_hbm,
       part_hbm, deg_hbm,
       sidx, didx, *rest) = refs
      bufs = rest[:NBUF]
      acc = rest[NBUF]
      gsems = rest[NBUF + 1:2 * NBUF + 1]
      ssems = rest[2 * NBUF + 1:3 * NBUF + 1]
      ones_v, dacc = rest[3 * NBUF + 1:]
    else:
      (p_hbm, ei_hbm, z_hbm,
       part_hbm,
       sidx, didx, *rest) = refs
      bufs = rest[:NBUF]
      acc = rest[NBUF]
      gsems = rest[NBUF + 1:2 * NBUF + 1]
      ssems = rest[2 * NBUF + 1:3 * NBUF + 1]
    c = lax.axis_index("c")
    s = lax.axis_index("s")
    r0 = s * ROWS_A

    def tile_slices(fn):
      @pl.when(s < NS - 1)
      def _():
        fn(r0, ROWS_A)

      @pl.when(s == NS - 1)
      def _():
        fn(ROWS_A * (NS - 1), ROWS_LAST)

    tile_slices(lambda o, n: pltpu.sync_copy(z_hbm.at[pl.ds(o, n)],
                                             acc.at[pl.ds(o, n)]))
    if with_deg:
      tile_slices(lambda o, n: pltpu.sync_copy(z1_hbm.at[pl.ds(o, n)],
                                               dacc.at[pl.ds(o, n)]))
      pltpu.sync_copy(ones_hbm, ones_v)
    plsc.subcore_barrier()

    nblk = BASE_BLKS + jnp.where(s < EXTRA, 1, 0)
    blk0 = BASE_BLKS * s + jnp.minimum(s, EXTRA)

    @pl.when(s < EXTRA)
    def _():
      pltpu.sync_copy(ei_hbm.at[0, pl.ds(blk0, MAXB)], sidx)
      pltpu.sync_copy(ei_hbm.at[1, pl.ds(blk0, MAXB)], didx)

    @pl.when(s >= EXTRA)
    def _():
      pltpu.sync_copy(ei_hbm.at[0, pl.ds(blk0, BASE_BLKS)],
                      sidx.at[pl.ds(0, BASE_BLKS)])
      pltpu.sync_copy(ei_hbm.at[1, pl.ds(blk0, BASE_BLKS)],
                      didx.at[pl.ds(0, BASE_BLKS)])

    def gather_start(g, b):
      for cc in range(NC):
        @pl.when(c == cc)
        def _(cc=cc):
          pltpu.async_copy(p_hbm.at[cc].at[sidx.at[g]], bufs[b], gsems[b])

    def gather_wait(g, b):
      for cc in range(NC):
        @pl.when(c == cc)
        def _(cc=cc):
          pltpu.make_async_copy(p_hbm.at[cc].at[sidx.at[g]], bufs[b],
                                gsems[b]).wait()

    def slot_has_deg(b):
      return with_deg and (b % 2 == 0)

    def scatter_start(g, b):
      pltpu.async_copy(bufs[b], acc.at[didx.at[g]], ssems[b], add=True)
      if slot_has_deg(b):
        for cc in range(NC):
          @pl.when(c == cc)
          def _(cc=cc):
            pltpu.async_copy(ones_v, dacc.at[didx.at[g + cc]], ssems[b],
                             add=True)

    def scatter_wait(g, b):
      pltpu.make_async_copy(bufs[b], acc.at[didx.at[g]], ssems[b]).wait()
      if slot_has_deg(b):
        pltpu.make_async_copy(ones_v, dacc.at[didx.at[g]], ssems[b]).wait()

    P = NBUF // 2
    NT = BASE_BLKS // NBUF
    for k in range(P):
      gather_start(k, k)

    def round_(j, carry):
      for u in range(NBUF):
        g = j * NBUF + u
        bpre = (u + P) % NBUF
        @pl.when((g + P >= NBUF) & (g + P < nblk))
        def _(g=g, bpre=bpre):
          scatter_wait(g + P - NBUF, bpre)

        @pl.when(g + P < nblk)
        def _(g=g, bpre=bpre):
          gather_start(g + P, bpre)

        gather_wait(g, u)
        scatter_start(g, u)
      return carry

    lax.fori_loop(0, NT, round_, 0)

    gt = BASE_BLKS
    bt = BASE_BLKS % NBUF

    @pl.when(s < EXTRA)
    def _():
      scatter_wait(gt - NBUF + P, (gt + P) % NBUF)
      gather_wait(gt, bt)
      pltpu.async_copy(bufs[bt], acc.at[didx.at[gt]], ssems[bt], add=True)
      if with_deg:
        @pl.when(c == 0)
        def _():
          pltpu.async_copy(ones_v, dacc.at[didx.at[gt]], ssems[bt],
                           add=True)

    bw = (gt + P) % NBUF
    for k in range(NBUF):
      g_std = BASE_BLKS - NBUF + k
      if k == bw:
        @pl.when(s >= EXTRA)
        def _(g_std=g_std, k=k):
          scatter_wait(g_std, k)
      elif k == bt:
        @pl.when(s >= EXTRA)
        def _(g_std=g_std, k=k):
          scatter_wait(g_std, k)

        @pl.when(s < EXTRA)
        def _(k=k):
          pltpu.make_async_copy(bufs[k], acc.at[didx.at[gt]],
                                ssems[k]).wait()
          if slot_has_deg(k):
            @pl.when(c == 0)
            def _(k=k):
              pltpu.make_async_copy(ones_v, dacc.at[didx.at[gt]],
                                    ssems[k]).wait()
      else:
        scatter_wait(g_std, k)

    plsc.subcore_barrier()

    tile_slices(lambda o, n: pltpu.sync_copy(acc.at[pl.ds(o, n)],
                                             part_hbm.at[c, pl.ds(o, n)]))
    if with_deg:
      tile_slices(lambda o, n: pltpu.sync_copy(dacc.at[pl.ds(o, n)],
                                               deg_hbm.at[c, pl.ds(o, n)]))

  params = pltpu.CompilerParams(use_tc_tiling_on_sc=False)
  return pl.kernel(body, out_type=tuple(out_type), mesh=_MESH,
                   scratch_types=scratch, compiler_params=params)


_segsum_deg = _make_segsum(HIDDEN // 2, True)
_segsum_l2 = _make_segsum(LATENT // 2, False)


ROWS_TC = 1000


def _tc_pre(x, Wl1a, Wl1b, Wr1):
  def body(x_ref, wla_ref, wlb_ref, wr_ref, p1_ref, r1_ref):
    xb = x_ref[...]
    dn = (((1,), (1,)), ((), ()))
    p1_ref[0] = lax.dot_general(xb, wla_ref[...], dn,
                                preferred_element_type=jnp.float32)
    p1_ref[1] = lax.dot_general(xb, wlb_ref[...], dn,
                                preferred_element_type=jnp.float32)
    r1_ref[...] = lax.dot_general(xb, wr_ref[...], dn,
                                  preferred_element_type=jnp.float32)
  grid = (N // ROWS_TC,)
  H2 = HIDDEN // 2
  return pl.pallas_call(
      body,
      grid=grid,
      in_specs=[
          pl.BlockSpec((ROWS_TC, IN_DIM), lambda i: (i, 0)),
          pl.BlockSpec((H2, IN_DIM), lambda i: (0, 0)),
          pl.BlockSpec((H2, IN_DIM), lambda i: (0, 0)),
          pl.BlockSpec((HIDDEN, IN_DIM), lambda i: (0, 0)),
      ],
      out_specs=[
          pl.BlockSpec((NC, ROWS_TC, H2), lambda i: (0, i, 0)),
          pl.BlockSpec((ROWS_TC, HIDDEN), lambda i: (i, 0)),
      ],
      out_shape=[
          jax.ShapeDtypeStruct((NC, N, H2), jnp.float32),
          jax.ShapeDtypeStruct((N, HIDDEN), jnp.float32),
      ],
  )(x, Wl1a, Wl1b, Wr1)


def _tc_mid(s1p, deg, bl1, r1, Wl2a, Wl2b, Wr2):
  H2 = HIDDEN // 2
  L2 = LATENT // 2

  def body(s1p_ref, deg_ref, bl1_ref, r1_ref, wla_ref, wlb_ref, wr_ref,
           p2_ref, r2_ref):
    ssum = jnp.concatenate([s1p_ref[0], s1p_ref[1]], axis=1)
    d = jnp.maximum(deg_ref[0, :, 0:1] + deg_ref[1, :, 0:1], 1.0)
    h = jnp.maximum(ssum / d + bl1_ref[...] + r1_ref[...], 0.0)
    dn = (((1,), (1,)), ((), ()))
    p2_ref[0] = lax.dot_general(h, wla_ref[...], dn,
                                preferred_element_type=jnp.float32)
    p2_ref[1] = lax.dot_general(h, wlb_ref[...], dn,
                                preferred_element_type=jnp.float32)
    r2_ref[...] = lax.dot_general(h, wr_ref[...], dn,
                                  preferred_element_type=jnp.float32)
  grid = (N // ROWS_TC,)
  return pl.pallas_call(
      body,
      grid=grid,
      in_specs=[
          pl.BlockSpec((NC, ROWS_TC, H2), lambda i: (0, i, 0)),
          pl.BlockSpec((NC, ROWS_TC, 16), lambda i: (0, i, 0)),
          pl.BlockSpec((1, HIDDEN), lambda i: (0, 0)),
          pl.BlockSpec((ROWS_TC, HIDDEN), lambda i: (i, 0)),
          pl.BlockSpec((L2, HIDDEN), lambda i: (0, 0)),
          pl.BlockSpec((L2, HIDDEN), lambda i: (0, 0)),
          pl.BlockSpec((LATENT, HIDDEN), lambda i: (0, 0)),
      ],
      out_specs=[
          pl.BlockSpec((NC, ROWS_TC, L2), lambda i: (0, i, 0)),
          pl.BlockSpec((ROWS_TC, LATENT), lambda i: (i, 0)),
      ],
      out_shape=[
          jax.ShapeDtypeStruct((NC, N, L2), jnp.float32),
          jax.ShapeDtypeStruct((N, LATENT), jnp.float32),
      ],
  )(s1p, deg, bl1, r1, Wl2a, Wl2b, Wr2)


def _tc_post(s2p, deg, bl2, r2, Wd, bd):
  L2 = LATENT // 2

  def body(s2p_ref, deg_ref, bl2_ref, r2_ref, wd_ref, bd_ref,
           z_ref, xh_ref):
    ssum = jnp.concatenate([s2p_ref[0], s2p_ref[1]], axis=1)
    d = jnp.maximum(deg_ref[0, :, 0:1] + deg_ref[1, :, 0:1], 1.0)
    z = ssum / d + bl2_ref[...] + r2_ref[...]
    z_ref[...] = z
    xh_ref[...] = lax.dot_general(z, wd_ref[...], (((1,), (1,)), ((), ())),
                                  preferred_element_type=jnp.float32) + bd_ref[...]
  grid = (N // ROWS_TC,)
  return pl.pallas_call(
      body,
      grid=grid,
      in_specs=[
          pl.BlockSpec((NC, ROWS_TC, L2), lambda i: (0, i, 0)),
          pl.BlockSpec((NC, ROWS_TC, 16), lambda i: (0, i, 0)),
          pl.BlockSpec((1, LATENT), lambda i: (0, 0)),
          pl.BlockSpec((ROWS_TC, LATENT), lambda i: (i, 0)),
          pl.BlockSpec((IN_DIM, LATENT), lambda i: (0, 0)),
          pl.BlockSpec((1, IN_DIM), lambda i: (0, 0)),
      ],
      out_specs=[
          pl.BlockSpec((ROWS_TC, LATENT), lambda i: (i, 0)),
          pl.BlockSpec((ROWS_TC, IN_DIM), lambda i: (i, 0)),
      ],
      out_shape=[
          jax.ShapeDtypeStruct((N, LATENT), jnp.float32),
          jax.ShapeDtypeStruct((N, IN_DIM), jnp.float32),
      ],
  )(s2p, deg, bl2, r2, Wd, bd)


def kernel(x, edge_index, Wl1, bl1, Wr1, Wl2, bl2, Wr2, Wd, bd):
  ei = edge_index.astype(jnp.int32).reshape(2, NBLKS, BLK)

  H2 = HIDDEN // 2
  L2 = LATENT // 2
  ones16 = jnp.zeros((BLK, 16), jnp.float32).at[:, 0].set(1.0)
  zh = jnp.zeros((N, H2), jnp.float32)
  zl = jnp.zeros((N, L2), jnp.float32)
  z16 = jnp.zeros((N, 16), jnp.float32)

  p1, r1 = _tc_pre(x, Wl1[:H2], Wl1[H2:], Wr1)
  s1p, deg = _segsum_deg(p1, ei, zh, ones16, z16)
  p2, r2 = _tc_mid(s1p, deg, bl1.reshape(1, HIDDEN), r1,
                   Wl2[:L2], Wl2[L2:], Wr2)
  (s2p,) = _segsum_l2(p2, ei, zl)
  z, x_hat = _tc_post(s2p, deg, bl2.reshape(1, LATENT), r2, Wd,
                      bd.reshape(1, IN_DIM))
  return (z, x_hat)


# speedup vs baseline: 1.0219x; 1.0219x over previous
import jax
import jax.numpy as jnp
from jax import lax
from jax.experimental import pallas as pl
from jax.experimental.pallas import tpu as pltpu
from jax.experimental.pallas import tpu_sc as plsc

N = 10000
E = 320000
IN_DIM = 128
HIDDEN = 128
LATENT = 64

NC = 2
NS = 16
BLK = 128
NBLKS = E // BLK
BASE_BLKS = NBLKS // NS
EXTRA = NBLKS - BASE_BLKS * NS
MAXB = BASE_BLKS + 1
ROWS_A = 624
ROWS_LAST = N - ROWS_A * (NS - 1)

_MESH = plsc.VectorSubcoreMesh(
    core_axis_name="c", subcore_axis_name="s", num_cores=NC, num_subcores=NS)


def _make_segsum(DH, with_deg):
  NBUF = 4
  out_type = [jax.ShapeDtypeStruct((NC, N, DH), jnp.float32)]
  if with_deg:
    out_type.append(jax.ShapeDtypeStruct((NC, N, 16), jnp.float32))
  scratch = [
      pltpu.VMEM((MAXB, BLK), jnp.int32),
      pltpu.VMEM((MAXB, BLK), jnp.int32),
  ]
  scratch += [pltpu.VMEM((BLK, DH), jnp.float32) for _ in range(NBUF)]
  scratch += [
      pltpu.VMEM_SHARED((N, DH), jnp.float32),
  ]
  scratch += [pltpu.SemaphoreType.DMA for _ in range(2 * NBUF)]
  if with_deg:
    scratch += [
        pltpu.VMEM((BLK, 16), jnp.float32),
        pltpu.VMEM_SHARED((N, 16), jnp.float32),
    ]

  def body(*refs):
    if with_deg:
      (p_hbm, ei_hbm, z_hbm, ones_hbm, z1_hbm,
       part_hbm, deg_hbm,
       sidx, didx, *rest) = refs
      bufs = rest[:NBUF]
      acc = rest[NBUF]
      gsems = rest[NBUF + 1:2 * NBUF + 1]
      ssems = rest[2 * NBUF + 1:3 * NBUF + 1]
      ones_v, dacc = rest[3 * NBUF + 1:]
    else:
      (p_hbm, ei_hbm, z_hbm,
       part_hbm,
       sidx, didx, *rest) = refs
      bufs = rest[:NBUF]
      acc = rest[NBUF]
      gsems = rest[NBUF + 1:2 * NBUF + 1]
      ssems = rest[2 * NBUF + 1:3 * NBUF + 1]
    c = lax.axis_index("c")
    s = lax.axis_index("s")
    r0 = s * ROWS_A

    def tile_slices(fn):
      @pl.when(s < NS - 1)
      def _():
        fn(r0, ROWS_A)

      @pl.when(s == NS - 1)
      def _():
        fn(ROWS_A * (NS - 1), ROWS_LAST)

    tile_slices(lambda o, n: pltpu.sync_copy(z_hbm.at[pl.ds(o, n)],
                                             acc.at[pl.ds(o, n)]))
    if with_deg:
      tile_slices(lambda o, n: pltpu.sync_copy(z1_hbm.at[pl.ds(o, n)],
                                               dacc.at[pl.ds(o, n)]))
      pltpu.sync_copy(ones_hbm, ones_v)
    plsc.subcore_barrier()

    nblk = BASE_BLKS + jnp.where(s < EXTRA, 1, 0)
    blk0 = BASE_BLKS * s + jnp.minimum(s, EXTRA)

    @pl.when(s < EXTRA)
    def _():
      pltpu.sync_copy(ei_hbm.at[0, pl.ds(blk0, MAXB)], sidx)
      pltpu.sync_copy(ei_hbm.at[1, pl.ds(blk0, MAXB)], didx)

    @pl.when(s >= EXTRA)
    def _():
      pltpu.sync_copy(ei_hbm.at[0, pl.ds(blk0, BASE_BLKS)],
                      sidx.at[pl.ds(0, BASE_BLKS)])
      pltpu.sync_copy(ei_hbm.at[1, pl.ds(blk0, BASE_BLKS)],
                      didx.at[pl.ds(0, BASE_BLKS)])

    def gather_start(g, b):
      for cc in range(NC):
        @pl.when(c == cc)
        def _(cc=cc):
          pltpu.async_copy(p_hbm.at[cc].at[sidx.at[g]], bufs[b], gsems[b])

    def gather_wait(g, b):
      for cc in range(NC):
        @pl.when(c == cc)
        def _(cc=cc):
          pltpu.make_async_copy(p_hbm.at[cc].at[sidx.at[g]], bufs[b],
                                gsems[b]).wait()

    def slot_has_deg(b):
      return with_deg and (b % 2 == 0)

    def scatter_start(g, b):
      pltpu.async_copy(bufs[b], acc.at[didx.at[g]], ssems[b], add=True)
      if slot_has_deg(b):
        for cc in range(NC):
          @pl.when(c == cc)
          def _(cc=cc):
            pltpu.async_copy(ones_v, dacc.at[didx.at[g + cc]], ssems[b],
                             add=True)

    def scatter_wait(g, b):
      pltpu.make_async_copy(bufs[b], acc.at[didx.at[g]], ssems[b]).wait()
      if slot_has_deg(b):
        pltpu.make_async_copy(ones_v, dacc.at[didx.at[g]], ssems[b]).wait()

    P = NBUF // 2
    NT = BASE_BLKS // NBUF
    for k in range(P):
      gather_start(k, k)

    def round_(j, carry):
      for u in range(NBUF):
        g = j * NBUF + u
        bpre = (u + P) % NBUF
        @pl.when((g + P >= NBUF) & (g + P < nblk))
        def _(g=g, bpre=bpre):
          scatter_wait(g + P - NBUF, bpre)

        @pl.when(g + P < nblk)
        def _(g=g, bpre=bpre):
          gather_start(g + P, bpre)

        gather_wait(g, u)
        scatter_start(g, u)
      return carry

    lax.fori_loop(0, NT, round_, 0)

    gt = BASE_BLKS
    bt = BASE_BLKS % NBUF

    @pl.when(s < EXTRA)
    def _():
      scatter_wait(gt - NBUF + P, (gt + P) % NBUF)
      gather_wait(gt, bt)
      pltpu.async_copy(bufs[bt], acc.at[didx.at[gt]], ssems[bt], add=True)
      if with_deg:
        @pl.when(c == 0)
        def _():
          pltpu.async_copy(ones_v, dacc.at[didx.at[gt]], ssems[bt],
                           add=True)

    bw = (gt + P) % NBUF
    for k in range(NBUF):
      g_std = BASE_BLKS - NBUF + k
      if k == bw:
        @pl.when(s >= EXTRA)
        def _(g_std=g_std, k=k):
          scatter_wait(g_std, k)
      elif k == bt:
        @pl.when(s >= EXTRA)
        def _(g_std=g_std, k=k):
          scatter_wait(g_std, k)

        @pl.when(s < EXTRA)
        def _(k=k):
          pltpu.make_async_copy(bufs[k], acc.at[didx.at[gt]],
                                ssems[k]).wait()
          if slot_has_deg(k):
            @pl.when(c == 0)
            def _(k=k):
              pltpu.make_async_copy(ones_v, dacc.at[didx.at[gt]],
                                    ssems[k]).wait()
      else:
        scatter_wait(g_std, k)

    plsc.subcore_barrier()

    tile_slices(lambda o, n: pltpu.sync_copy(acc.at[pl.ds(o, n)],
                                             part_hbm.at[c, pl.ds(o, n)]))
    if with_deg:
      tile_slices(lambda o, n: pltpu.sync_copy(dacc.at[pl.ds(o, n)],
                                               deg_hbm.at[c, pl.ds(o, n)]))

  params = pltpu.CompilerParams(use_tc_tiling_on_sc=False)
  return pl.kernel(body, out_type=tuple(out_type), mesh=_MESH,
                   scratch_types=scratch, compiler_params=params)


_segsum_deg = _make_segsum(HIDDEN // 2, True)
_segsum_l2 = _make_segsum(LATENT // 2, False)


ROWS_TC = 2000


def _tc_pre(x, Wl1a, Wl1b, Wr1):
  def body(x_ref, wla_ref, wlb_ref, wr_ref, p1_ref, r1_ref):
    xb = x_ref[...]
    dn = (((1,), (1,)), ((), ()))
    p1_ref[0] = lax.dot_general(xb, wla_ref[...], dn,
                                preferred_element_type=jnp.float32)
    p1_ref[1] = lax.dot_general(xb, wlb_ref[...], dn,
                                preferred_element_type=jnp.float32)
    r1_ref[...] = lax.dot_general(xb, wr_ref[...], dn,
                                  preferred_element_type=jnp.float32)
  grid = (N // ROWS_TC,)
  H2 = HIDDEN // 2
  return pl.pallas_call(
      body,
      grid=grid,
      in_specs=[
          pl.BlockSpec((ROWS_TC, IN_DIM), lambda i: (i, 0)),
          pl.BlockSpec((H2, IN_DIM), lambda i: (0, 0)),
          pl.BlockSpec((H2, IN_DIM), lambda i: (0, 0)),
          pl.BlockSpec((HIDDEN, IN_DIM), lambda i: (0, 0)),
      ],
      out_specs=[
          pl.BlockSpec((NC, ROWS_TC, H2), lambda i: (0, i, 0)),
          pl.BlockSpec((ROWS_TC, HIDDEN), lambda i: (i, 0)),
      ],
      out_shape=[
          jax.ShapeDtypeStruct((NC, N, H2), jnp.float32),
          jax.ShapeDtypeStruct((N, HIDDEN), jnp.float32),
      ],
  )(x, Wl1a, Wl1b, Wr1)


def _tc_mid(s1p, deg, bl1, r1, Wl2a, Wl2b, Wr2):
  H2 = HIDDEN // 2
  L2 = LATENT // 2

  def body(s1p_ref, deg_ref, bl1_ref, r1_ref, wla_ref, wlb_ref, wr_ref,
           p2_ref, r2_ref):
    ssum = jnp.concatenate([s1p_ref[0], s1p_ref[1]], axis=1)
    d = jnp.maximum(deg_ref[0, :, 0:1] + deg_ref[1, :, 0:1], 1.0)
    h = jnp.maximum(ssum / d + bl1_ref[...] + r1_ref[...], 0.0)
    dn = (((1,), (1,)), ((), ()))
    p2_ref[0] = lax.dot_general(h, wla_ref[...], dn,
                                preferred_element_type=jnp.float32)
    p2_ref[1] = lax.dot_general(h, wlb_ref[...], dn,
                                preferred_element_type=jnp.float32)
    r2_ref[...] = lax.dot_general(h, wr_ref[...], dn,
                                  preferred_element_type=jnp.float32)
  grid = (N // ROWS_TC,)
  return pl.pallas_call(
      body,
      grid=grid,
      in_specs=[
          pl.BlockSpec((NC, ROWS_TC, H2), lambda i: (0, i, 0)),
          pl.BlockSpec((NC, ROWS_TC, 16), lambda i: (0, i, 0)),
          pl.BlockSpec((1, HIDDEN), lambda i: (0, 0)),
          pl.BlockSpec((ROWS_TC, HIDDEN), lambda i: (i, 0)),
          pl.BlockSpec((L2, HIDDEN), lambda i: (0, 0)),
          pl.BlockSpec((L2, HIDDEN), lambda i: (0, 0)),
          pl.BlockSpec((LATENT, HIDDEN), lambda i: (0, 0)),
      ],
      out_specs=[
          pl.BlockSpec((NC, ROWS_TC, L2), lambda i: (0, i, 0)),
          pl.BlockSpec((ROWS_TC, LATENT), lambda i: (i, 0)),
      ],
      out_shape=[
          jax.ShapeDtypeStruct((NC, N, L2), jnp.float32),
          jax.ShapeDtypeStruct((N, LATENT), jnp.float32),
      ],
  )(s1p, deg, bl1, r1, Wl2a, Wl2b, Wr2)


def _tc_post(s2p, deg, bl2, r2, Wd, bd):
  L2 = LATENT // 2

  def body(s2p_ref, deg_ref, bl2_ref, r2_ref, wd_ref, bd_ref,
           z_ref, xh_ref):
    ssum = jnp.concatenate([s2p_ref[0], s2p_ref[1]], axis=1)
    d = jnp.maximum(deg_ref[0, :, 0:1] + deg_ref[1, :, 0:1], 1.0)
    z = ssum / d + bl2_ref[...] + r2_ref[...]
    z_ref[...] = z
    xh_ref[...] = lax.dot_general(z, wd_ref[...], (((1,), (1,)), ((), ())),
                                  preferred_element_type=jnp.float32) + bd_ref[...]
  grid = (N // ROWS_TC,)
  return pl.pallas_call(
      body,
      grid=grid,
      in_specs=[
          pl.BlockSpec((NC, ROWS_TC, L2), lambda i: (0, i, 0)),
          pl.BlockSpec((NC, ROWS_TC, 16), lambda i: (0, i, 0)),
          pl.BlockSpec((1, LATENT), lambda i: (0, 0)),
          pl.BlockSpec((ROWS_TC, LATENT), lambda i: (i, 0)),
          pl.BlockSpec((IN_DIM, LATENT), lambda i: (0, 0)),
          pl.BlockSpec((1, IN_DIM), lambda i: (0, 0)),
      ],
      out_specs=[
          pl.BlockSpec((ROWS_TC, LATENT), lambda i: (i, 0)),
          pl.BlockSpec((ROWS_TC, IN_DIM), lambda i: (i, 0)),
      ],
      out_shape=[
          jax.ShapeDtypeStruct((N, LATENT), jnp.float32),
          jax.ShapeDtypeStruct((N, IN_DIM), jnp.float32),
      ],
  )(s2p, deg, bl2, r2, Wd, bd)


def kernel(x, edge_index, Wl1, bl1, Wr1, Wl2, bl2, Wr2, Wd, bd):
  ei = edge_index.astype(jnp.int32).reshape(2, NBLKS, BLK)

  H2 = HIDDEN // 2
  L2 = LATENT // 2
  ones16 = jnp.zeros((BLK, 16), jnp.float32).at[:, 0].set(1.0)
  zh = jnp.zeros((N, H2), jnp.float32)
  zl = jnp.zeros((N, L2), jnp.float32)
  z16 = jnp.zeros((N, 16), jnp.float32)

  p1, r1 = _tc_pre(x, Wl1[:H2], Wl1[H2:], Wr1)
  s1p, deg = _segsum_deg(p1, ei, zh, ones16, z16)
  p2, r2 = _tc_mid(s1p, deg, bl1.reshape(1, HIDDEN), r1,
                   Wl2[:L2], Wl2[L2:], Wr2)
  (s2p,) = _segsum_l2(p2, ei, zl)
  z, x_hat = _tc_post(s2p, deg, bl2.reshape(1, LATENT), r2, Wd,
                      bd.reshape(1, IN_DIM))
  return (z, x_hat)


# speedup vs baseline: 1.0305x; 1.0084x over previous
import jax
import jax.numpy as jnp
from jax import lax
from jax.experimental import pallas as pl
from jax.experimental.pallas import tpu as pltpu
from jax.experimental.pallas import tpu_sc as plsc

N = 10000
E = 320000
IN_DIM = 128
HIDDEN = 128
LATENT = 64

NC = 2
NS = 16
BLK = 128
NBLKS = E // BLK
BASE_BLKS = NBLKS // NS
EXTRA = NBLKS - BASE_BLKS * NS
MAXB = BASE_BLKS + 1
ROWS_A = 624
ROWS_LAST = N - ROWS_A * (NS - 1)

_MESH = plsc.VectorSubcoreMesh(
    core_axis_name="c", subcore_axis_name="s", num_cores=NC, num_subcores=NS)


def _make_segsum(DH, with_deg):
  NBUF = 4
  out_type = [jax.ShapeDtypeStruct((NC, N, DH), jnp.float32)]
  if with_deg:
    out_type.append(jax.ShapeDtypeStruct((NC, N, 16), jnp.float32))
  scratch = [
      pltpu.VMEM((MAXB, BLK), jnp.int32),
      pltpu.VMEM((MAXB, BLK), jnp.int32),
  ]
  scratch += [pltpu.VMEM((BLK, DH), jnp.float32) for _ in range(NBUF)]
  scratch += [
      pltpu.VMEM_SHARED((N, DH), jnp.float32),
  ]
  scratch += [pltpu.SemaphoreType.DMA for _ in range(2 * NBUF)]
  if with_deg:
    scratch += [
        pltpu.VMEM((BLK, 16), jnp.float32),
        pltpu.VMEM_SHARED((N, 16), jnp.float32),
    ]

  def body(*refs):
    if with_deg:
      (p_hbm, ei_hbm, z_hbm, ones_hbm, z1_hbm,
       part_hbm, deg_hbm,
       sidx, didx, *rest) = refs
      bufs = rest[:NBUF]
      acc = rest[NBUF]
      gsems = rest[NBUF + 1:2 * NBUF + 1]
      ssems = rest[2 * NBUF + 1:3 * NBUF + 1]
      ones_v, dacc = rest[3 * NBUF + 1:]
    else:
      (p_hbm, ei_hbm, z_hbm,
       part_hbm,
       sidx, didx, *rest) = refs
      bufs = rest[:NBUF]
      acc = rest[NBUF]
      gsems = rest[NBUF + 1:2 * NBUF + 1]
      ssems = rest[2 * NBUF + 1:3 * NBUF + 1]
    c = lax.axis_index("c")
    s = lax.axis_index("s")
    r0 = s * ROWS_A

    def tile_slices(fn):
      @pl.when(s < NS - 1)
      def _():
        fn(r0, ROWS_A)

      @pl.when(s == NS - 1)
      def _():
        fn(ROWS_A * (NS - 1), ROWS_LAST)

    tile_slices(lambda o, n: pltpu.sync_copy(z_hbm.at[pl.ds(o, n)],
                                             acc.at[pl.ds(o, n)]))
    if with_deg:
      tile_slices(lambda o, n: pltpu.sync_copy(z1_hbm.at[pl.ds(o, n)],
                                               dacc.at[pl.ds(o, n)]))
      pltpu.sync_copy(ones_hbm, ones_v)
    plsc.subcore_barrier()

    nblk = BASE_BLKS + jnp.where(s < EXTRA, 1, 0)
    blk0 = BASE_BLKS * s + jnp.minimum(s, EXTRA)

    @pl.when(s < EXTRA)
    def _():
      pltpu.sync_copy(ei_hbm.at[0, pl.ds(blk0, MAXB)], sidx)
      pltpu.sync_copy(ei_hbm.at[1, pl.ds(blk0, MAXB)], didx)

    @pl.when(s >= EXTRA)
    def _():
      pltpu.sync_copy(ei_hbm.at[0, pl.ds(blk0, BASE_BLKS)],
                      sidx.at[pl.ds(0, BASE_BLKS)])
      pltpu.sync_copy(ei_hbm.at[1, pl.ds(blk0, BASE_BLKS)],
                      didx.at[pl.ds(0, BASE_BLKS)])

    def gather_start(g, b):
      for cc in range(NC):
        @pl.when(c == cc)
        def _(cc=cc):
          pltpu.async_copy(p_hbm.at[cc].at[sidx.at[g]], bufs[b], gsems[b])

    def gather_wait(g, b):
      for cc in range(NC):
        @pl.when(c == cc)
        def _(cc=cc):
          pltpu.make_async_copy(p_hbm.at[cc].at[sidx.at[g]], bufs[b],
                                gsems[b]).wait()

    def slot_has_deg(b):
      return with_deg and (b % 2 == 0)

    def scatter_start(g, b):
      pltpu.async_copy(bufs[b], acc.at[didx.at[g]], ssems[b], add=True)
      if slot_has_deg(b):
        for cc in range(NC):
          @pl.when(c == cc)
          def _(cc=cc):
            pltpu.async_copy(ones_v, dacc.at[didx.at[g + cc]], ssems[b],
                             add=True)

    def scatter_wait(g, b):
      pltpu.make_async_copy(bufs[b], acc.at[didx.at[g]], ssems[b]).wait()
      if slot_has_deg(b):
        pltpu.make_async_copy(ones_v, dacc.at[didx.at[g]], ssems[b]).wait()

    P = NBUF // 2
    NT = BASE_BLKS // NBUF
    for k in range(P):
      gather_start(k, k)

    def round_(j, carry):
      for u in range(NBUF):
        g = j * NBUF + u
        bpre = (u + P) % NBUF
        @pl.when((g + P >= NBUF) & (g + P < nblk))
        def _(g=g, bpre=bpre):
          scatter_wait(g + P - NBUF, bpre)

        @pl.when(g + P < nblk)
        def _(g=g, bpre=bpre):
          gather_start(g + P, bpre)

        gather_wait(g, u)
        scatter_start(g, u)
      return carry

    lax.fori_loop(0, NT, round_, 0)

    gt = BASE_BLKS
    bt = BASE_BLKS % NBUF

    @pl.when(s < EXTRA)
    def _():
      scatter_wait(gt - NBUF + P, (gt + P) % NBUF)
      gather_wait(gt, bt)
      pltpu.async_copy(bufs[bt], acc.at[didx.at[gt]], ssems[bt], add=True)
      if with_deg:
        @pl.when(c == 0)
        def _():
          pltpu.async_copy(ones_v, dacc.at[didx.at[gt]], ssems[bt],
                           add=True)

    bw = (gt + P) % NBUF
    for k in range(NBUF):
      g_std = BASE_BLKS - NBUF + k
      if k == bw:
        @pl.when(s >= EXTRA)
        def _(g_std=g_std, k=k):
          scatter_wait(g_std, k)
      elif k == bt:
        @pl.when(s >= EXTRA)
        def _(g_std=g_std, k=k):
          scatter_wait(g_std, k)

        @pl.when(s < EXTRA)
        def _(k=k):
          pltpu.make_async_copy(bufs[k], acc.at[didx.at[gt]],
                                ssems[k]).wait()
          if slot_has_deg(k):
            @pl.when(c == 0)
            def _(k=k):
              pltpu.make_async_copy(ones_v, dacc.at[didx.at[gt]],
                                    ssems[k]).wait()
      else:
        scatter_wait(g_std, k)

    plsc.subcore_barrier()

    tile_slices(lambda o, n: pltpu.sync_copy(acc.at[pl.ds(o, n)],
                                             part_hbm.at[c, pl.ds(o, n)]))
    if with_deg:
      tile_slices(lambda o, n: pltpu.sync_copy(dacc.at[pl.ds(o, n)],
                                               deg_hbm.at[c, pl.ds(o, n)]))

  params = pltpu.CompilerParams(use_tc_tiling_on_sc=False)
  return pl.kernel(body, out_type=tuple(out_type), mesh=_MESH,
                   scratch_types=scratch, compiler_params=params)


_segsum_deg = _make_segsum(HIDDEN // 2, True)
_segsum_l2 = _make_segsum(LATENT // 2, False)


ROWS_TC = 5000


def _tc_pre(x, Wl1a, Wl1b, Wr1):
  def body(x_ref, wla_ref, wlb_ref, wr_ref, p1_ref, r1_ref):
    xb = x_ref[...]
    dn = (((1,), (1,)), ((), ()))
    p1_ref[0] = lax.dot_general(xb, wla_ref[...], dn,
                                preferred_element_type=jnp.float32)
    p1_ref[1] = lax.dot_general(xb, wlb_ref[...], dn,
                                preferred_element_type=jnp.float32)
    r1_ref[...] = lax.dot_general(xb, wr_ref[...], dn,
                                  preferred_element_type=jnp.float32)
  grid = (N // ROWS_TC,)
  H2 = HIDDEN // 2
  return pl.pallas_call(
      body,
      grid=grid,
      in_specs=[
          pl.BlockSpec((ROWS_TC, IN_DIM), lambda i: (i, 0)),
          pl.BlockSpec((H2, IN_DIM), lambda i: (0, 0)),
          pl.BlockSpec((H2, IN_DIM), lambda i: (0, 0)),
          pl.BlockSpec((HIDDEN, IN_DIM), lambda i: (0, 0)),
      ],
      out_specs=[
          pl.BlockSpec((NC, ROWS_TC, H2), lambda i: (0, i, 0)),
          pl.BlockSpec((ROWS_TC, HIDDEN), lambda i: (i, 0)),
      ],
      out_shape=[
          jax.ShapeDtypeStruct((NC, N, H2), jnp.float32),
          jax.ShapeDtypeStruct((N, HIDDEN), jnp.float32),
      ],
  )(x, Wl1a, Wl1b, Wr1)


def _tc_mid(s1p, deg, bl1, r1, Wl2a, Wl2b, Wr2):
  H2 = HIDDEN // 2
  L2 = LATENT // 2

  def body(s1p_ref, deg_ref, bl1_ref, r1_ref, wla_ref, wlb_ref, wr_ref,
           p2_ref, r2_ref):
    ssum = jnp.concatenate([s1p_ref[0], s1p_ref[1]], axis=1)
    d = jnp.maximum(deg_ref[0, :, 0:1] + deg_ref[1, :, 0:1], 1.0)
    h = jnp.maximum(ssum / d + bl1_ref[...] + r1_ref[...], 0.0)
    dn = (((1,), (1,)), ((), ()))
    p2_ref[0] = lax.dot_general(h, wla_ref[...], dn,
                                preferred_element_type=jnp.float32)
    p2_ref[1] = lax.dot_general(h, wlb_ref[...], dn,
                                preferred_element_type=jnp.float32)
    r2_ref[...] = lax.dot_general(h, wr_ref[...], dn,
                                  preferred_element_type=jnp.float32)
  grid = (N // ROWS_TC,)
  return pl.pallas_call(
      body,
      grid=grid,
      in_specs=[
          pl.BlockSpec((NC, ROWS_TC, H2), lambda i: (0, i, 0)),
          pl.BlockSpec((NC, ROWS_TC, 16), lambda i: (0, i, 0)),
          pl.BlockSpec((1, HIDDEN), lambda i: (0, 0)),
          pl.BlockSpec((ROWS_TC, HIDDEN), lambda i: (i, 0)),
          pl.BlockSpec((L2, HIDDEN), lambda i: (0, 0)),
          pl.BlockSpec((L2, HIDDEN), lambda i: (0, 0)),
          pl.BlockSpec((LATENT, HIDDEN), lambda i: (0, 0)),
      ],
      out_specs=[
          pl.BlockSpec((NC, ROWS_TC, L2), lambda i: (0, i, 0)),
          pl.BlockSpec((ROWS_TC, LATENT), lambda i: (i, 0)),
      ],
      out_shape=[
          jax.ShapeDtypeStruct((NC, N, L2), jnp.float32),
          jax.ShapeDtypeStruct((N, LATENT), jnp.float32),
      ],
  )(s1p, deg, bl1, r1, Wl2a, Wl2b, Wr2)


def _tc_post(s2p, deg, bl2, r2, Wd, bd):
  L2 = LATENT // 2

  def body(s2p_ref, deg_ref, bl2_ref, r2_ref, wd_ref, bd_ref,
           z_ref, xh_ref):
    ssum = jnp.concatenate([s2p_ref[0], s2p_ref[1]], axis=1)
    d = jnp.maximum(deg_ref[0, :, 0:1] + deg_ref[1, :, 0:1], 1.0)
    z = ssum / d + bl2_ref[...] + r2_ref[...]
    z_ref[...] = z
    xh_ref[...] = lax.dot_general(z, wd_ref[...], (((1,), (1,)), ((), ())),
                                  preferred_element_type=jnp.float32) + bd_ref[...]
  grid = (N // ROWS_TC,)
  return pl.pallas_call(
      body,
      grid=grid,
      in_specs=[
          pl.BlockSpec((NC, ROWS_TC, L2), lambda i: (0, i, 0)),
          pl.BlockSpec((NC, ROWS_TC, 16), lambda i: (0, i, 0)),
          pl.BlockSpec((1, LATENT), lambda i: (0, 0)),
          pl.BlockSpec((ROWS_TC, LATENT), lambda i: (i, 0)),
          pl.BlockSpec((IN_DIM, LATENT), lambda i: (0, 0)),
          pl.BlockSpec((1, IN_DIM), lambda i: (0, 0)),
      ],
      out_specs=[
          pl.BlockSpec((ROWS_TC, LATENT), lambda i: (i, 0)),
          pl.BlockSpec((ROWS_TC, IN_DIM), lambda i: (i, 0)),
      ],
      out_shape=[
          jax.ShapeDtypeStruct((N, LATENT), jnp.float32),
          jax.ShapeDtypeStruct((N, IN_DIM), jnp.float32),
      ],
  )(s2p, deg, bl2, r2, Wd, bd)


def kernel(x, edge_index, Wl1, bl1, Wr1, Wl2, bl2, Wr2, Wd, bd):
  ei = edge_index.astype(jnp.int32).reshape(2, NBLKS, BLK)

  H2 = HIDDEN // 2
  L2 = LATENT // 2
  ones16 = jnp.zeros((BLK, 16), jnp.float32).at[:, 0].set(1.0)
  zh = jnp.zeros((N, H2), jnp.float32)
  zl = jnp.zeros((N, L2), jnp.float32)
  z16 = jnp.zeros((N, 16), jnp.float32)

  p1, r1 = _tc_pre(x, Wl1[:H2], Wl1[H2:], Wr1)
  s1p, deg = _segsum_deg(p1, ei, zh, ones16, z16)
  p2, r2 = _tc_mid(s1p, deg, bl1.reshape(1, HIDDEN), r1,
                   Wl2[:L2], Wl2[L2:], Wr2)
  (s2p,) = _segsum_l2(p2, ei, zl)
  z, x_hat = _tc_post(s2p, deg, bl2.reshape(1, LATENT), r2, Wd,
                      bd.reshape(1, IN_DIM))
  return (z, x_hat)
